# baseline (speedup 1.0000x reference)
import functools

import jax
import jax.numpy as jnp
from jax import lax
from jax.experimental import pallas as pl
from jax.experimental.pallas import tpu as pltpu
from jax.experimental.pallas import tpu_sc as plsc

B = 4096
N_CATEGORY = 1000
N_DIM = 16

_INFO = plsc.get_sparse_core_info()
_NC, _NS, _L = _INFO.num_cores, _INFO.num_subcores, _INFO.num_lanes
_NW = _NC * _NS
_B_PER_W = B // _NW
_TASKS = N_DIM * _B_PER_W

_C_STRIDE = N_DIM * B
_TDIM_STRIDE = 8 * B
_ITILE_STRIDE = 8 * 128


def _make_gather_kernel():
    mesh = plsc.VectorSubcoreMesh(core_axis_name="c", subcore_axis_name="s")

    @functools.partial(
        pl.kernel,
        mesh=mesh,
        out_type=jax.ShapeDtypeStruct((2 * B * 8,), jnp.float32),
        scratch_types=[
            pltpu.VMEM((_B_PER_W,), jnp.int32),
            pltpu.VMEM((_B_PER_W,), jnp.int32),
            pltpu.VMEM((_TASKS,), jnp.int32),
            pltpu.VMEM((_TASKS,), jnp.float32),
            pltpu.SemaphoreType.DMA,
            pltpu.SemaphoreType.DMA,
            pltpu.SemaphoreType.DMA,
        ],
        compiler_params=pltpu.CompilerParams(use_tc_tiling_on_sc=False),
    )
    def gather_kernel(
        flat_hbm,
        label_hbm,
        out_hbm,
        lbl_v,
        col_v,
        idx_v,
        buf_v,
        sem_a,
        sem_b,
        sem_w,
    ):
        wid = lax.axis_index("s") * _NC + lax.axis_index("c")
        base = wid * _B_PER_W
        pltpu.sync_copy(label_hbm.at[pl.ds(base, _B_PER_W)], lbl_v)
        lane = lax.iota(jnp.int32, _L)
        col_base = wid * _ITILE_STRIDE
        for b in range(_B_PER_W // _L):
            col_v[pl.ds(b * _L, _L)] = (
                lbl_v[pl.ds(b * _L, _L)] * _C_STRIDE
                + (col_base + b * _L)
                + lane
            )
        copies = []
        for d in range(N_DIM):
            off = (d // 8) * _TDIM_STRIDE + (d % 8) * 128
            for b in range(_B_PER_W // _L):
                idx_v[pl.ds(d * _B_PER_W + b * _L, _L)] = (
                    col_v[pl.ds(b * _L, _L)] + off
                )
            copies.append(
                pltpu.async_copy(
                    flat_hbm.at[idx_v.at[pl.ds(d * _B_PER_W, _B_PER_W)]],
                    buf_v.at[pl.ds(d * _B_PER_W, _B_PER_W)],
                    sem_a if d < 8 else sem_b,
                )
            )
        writes = []
        for d in range(N_DIM):
            copies[d].wait()
            off = (d // 8) * _TDIM_STRIDE + (d % 8) * 128
            writes.append(
                pltpu.async_copy(
                    buf_v.at[pl.ds(d * _B_PER_W, _B_PER_W)],
                    out_hbm.at[pl.ds(off + col_base, _B_PER_W)],
                    sem_w,
                )
            )
        for w in writes:
            w.wait()

    return gather_kernel


_GATHER = _make_gather_kernel()


@jax.jit
def kernel(x, label):
    flat = (
        x.transpose(1, 2, 0)
        .reshape(N_CATEGORY, 2, 8, B // 128, 128)
        .transpose(0, 1, 3, 2, 4)
        .reshape(-1)
    )
    out_flat = _GATHER(flat, label.astype(jnp.int32))
    out = (
        out_flat.reshape(2, B // 128, 8, 128)
        .transpose(1, 3, 0, 2)
        .reshape(B, N_DIM)
    )
    return out


# speedup vs baseline: 1.0174x; 1.0174x over previous
import functools

import jax
import jax.numpy as jnp
from jax import lax
from jax.experimental import pallas as pl
from jax.experimental.pallas import tpu as pltpu
from jax.experimental.pallas import tpu_sc as plsc

B = 4096
N_CATEGORY = 1000
N_DIM = 16

_INFO = plsc.get_sparse_core_info()
_NC, _NS, _L = _INFO.num_cores, _INFO.num_subcores, _INFO.num_lanes
_NW = _NC * _NS
_B_PER_W = B // _NW
_TASKS = N_DIM * _B_PER_W

_C_STRIDE = N_DIM * B
_TDIM_STRIDE = 8 * B
_ITILE_STRIDE = 8 * 128


def _make_gather_kernel():
    mesh = plsc.VectorSubcoreMesh(core_axis_name="c", subcore_axis_name="s")

    @functools.partial(
        pl.kernel,
        mesh=mesh,
        out_type=jax.ShapeDtypeStruct((2 * B * 8,), jnp.float32),
        scratch_types=[
            pltpu.VMEM((_B_PER_W,), jnp.int32),
            pltpu.VMEM((_B_PER_W,), jnp.int32),
            pltpu.VMEM((_TASKS,), jnp.int32),
            pltpu.VMEM((_TASKS,), jnp.float32),
            pltpu.SemaphoreType.DMA,
            pltpu.SemaphoreType.DMA,
            pltpu.SemaphoreType.DMA,
        ],
        compiler_params=pltpu.CompilerParams(use_tc_tiling_on_sc=False),
    )
    def gather_kernel(
        flat_hbm,
        label_hbm,
        out_hbm,
        lbl_v,
        col_v,
        idx_v,
        buf_v,
        sem_a,
        sem_b,
        sem_w,
    ):
        wid = lax.axis_index("s") * _NC + lax.axis_index("c")
        base = wid * _B_PER_W
        pltpu.sync_copy(label_hbm.at[pl.ds(base, _B_PER_W)], lbl_v)
        lane = lax.iota(jnp.int32, _L)
        col_base = wid * _ITILE_STRIDE
        for b in range(_B_PER_W // _L):
            col_v[pl.ds(b * _L, _L)] = (
                lbl_v[pl.ds(b * _L, _L)] * _C_STRIDE
                + (col_base + b * _L)
                + lane
            )
        chunk_d = 4
        copies = []
        for d in range(N_DIM):
            off = (d // 8) * _TDIM_STRIDE + (d % 8) * 128
            for b in range(_B_PER_W // _L):
                idx_v[pl.ds(d * _B_PER_W + b * _L, _L)] = (
                    col_v[pl.ds(b * _L, _L)] + off
                )
            if d % chunk_d == chunk_d - 1:
                c0 = (d - chunk_d + 1) * _B_PER_W
                clen = chunk_d * _B_PER_W
                copies.append(
                    pltpu.async_copy(
                        flat_hbm.at[idx_v.at[pl.ds(c0, clen)]],
                        buf_v.at[pl.ds(c0, clen)],
                        sem_a if d < 8 else sem_b,
                    )
                )
        half = _TASKS // 2
        nhalf = len(copies) // 2
        for c in copies[:nhalf]:
            c.wait()
        w0 = pltpu.async_copy(
            buf_v.at[pl.ds(0, half)], out_hbm.at[pl.ds(wid * half, half)], sem_w
        )
        for c in copies[nhalf:]:
            c.wait()
        w1 = pltpu.async_copy(
            buf_v.at[pl.ds(half, half)],
            out_hbm.at[pl.ds(8 * B + wid * half, half)],
            sem_w,
        )
        w0.wait()
        w1.wait()

    return gather_kernel


_GATHER = _make_gather_kernel()


@jax.jit
def kernel(x, label):
    flat = (
        x.transpose(1, 2, 0)
        .reshape(N_CATEGORY, 2, 8, B // 128, 128)
        .transpose(0, 1, 3, 2, 4)
        .reshape(-1)
    )
    out_flat = _GATHER(flat, label.astype(jnp.int32))
    out = (
        out_flat.reshape(2, B // 128, 8, 128)
        .transpose(1, 3, 0, 2)
        .reshape(B, N_DIM)
    )
    return out


# speedup vs baseline: 1.0234x; 1.0059x over previous
import functools

import jax
import jax.numpy as jnp
from jax import lax
from jax.experimental import pallas as pl
from jax.experimental.pallas import tpu as pltpu
from jax.experimental.pallas import tpu_sc as plsc

B = 4096
N_CATEGORY = 1000
N_DIM = 16

_INFO = plsc.get_sparse_core_info()
_NC, _NS, _L = _INFO.num_cores, _INFO.num_subcores, _INFO.num_lanes
_NW = _NC * _NS
_B_PER_W = B // _NW
_TASKS = N_DIM * _B_PER_W

_C_STRIDE = N_DIM * B
_TDIM_STRIDE = 8 * B
_ITILE_STRIDE = 8 * 128


def _make_gather_kernel():
    mesh = plsc.VectorSubcoreMesh(core_axis_name="c", subcore_axis_name="s")

    @functools.partial(
        pl.kernel,
        mesh=mesh,
        out_type=jax.ShapeDtypeStruct((2 * B * 8,), jnp.float32),
        scratch_types=[
            pltpu.VMEM((_B_PER_W,), jnp.int32),
            pltpu.VMEM((_B_PER_W,), jnp.int32),
            pltpu.VMEM((_TASKS,), jnp.int32),
            pltpu.VMEM((_TASKS,), jnp.float32),
            pltpu.SemaphoreType.DMA,
            pltpu.SemaphoreType.DMA,
            pltpu.SemaphoreType.DMA,
        ],
        compiler_params=pltpu.CompilerParams(use_tc_tiling_on_sc=False),
    )
    def gather_kernel(
        flat_hbm,
        label_hbm,
        out_hbm,
        lbl_v,
        col_v,
        idx_v,
        buf_v,
        sem_a,
        sem_b,
        sem_w,
    ):
        wid = lax.axis_index("s") * _NC + lax.axis_index("c")
        base = wid * _B_PER_W
        pltpu.sync_copy(label_hbm.at[pl.ds(base, _B_PER_W)], lbl_v)
        lane = lax.iota(jnp.int32, _L)
        col_base = wid * _ITILE_STRIDE
        for b in range(_B_PER_W // _L):
            col_v[pl.ds(b * _L, _L)] = (
                lbl_v[pl.ds(b * _L, _L)] * _C_STRIDE
                + (col_base + b * _L)
                + lane
            )
        chunk_d = 8
        copies = []
        for d in range(N_DIM):
            off = (d // 8) * _TDIM_STRIDE + (d % 8) * 128
            for b in range(_B_PER_W // _L):
                idx_v[pl.ds(d * _B_PER_W + b * _L, _L)] = (
                    col_v[pl.ds(b * _L, _L)] + off
                )
            if d % chunk_d == chunk_d - 1:
                c0 = (d - chunk_d + 1) * _B_PER_W
                clen = chunk_d * _B_PER_W
                copies.append(
                    pltpu.async_copy(
                        flat_hbm.at[idx_v.at[pl.ds(c0, clen)]],
                        buf_v.at[pl.ds(c0, clen)],
                        sem_a if d < 8 else sem_b,
                    )
                )
        half = _TASKS // 2
        nhalf = len(copies) // 2
        for c in copies[:nhalf]:
            c.wait()
        w0 = pltpu.async_copy(
            buf_v.at[pl.ds(0, half)], out_hbm.at[pl.ds(wid * half, half)], sem_w
        )
        for c in copies[nhalf:]:
            c.wait()
        w1 = pltpu.async_copy(
            buf_v.at[pl.ds(half, half)],
            out_hbm.at[pl.ds(8 * B + wid * half, half)],
            sem_w,
        )
        w0.wait()
        w1.wait()

    return gather_kernel


_GATHER = _make_gather_kernel()


@jax.jit
def kernel(x, label):
    flat = (
        x.transpose(1, 2, 0)
        .reshape(N_CATEGORY, 2, 8, B // 128, 128)
        .transpose(0, 1, 3, 2, 4)
        .reshape(-1)
    )
    out_flat = _GATHER(flat, label.astype(jnp.int32))
    out = (
        out_flat.reshape(2, B // 128, 8, 128)
        .transpose(1, 3, 0, 2)
        .reshape(B, N_DIM)
    )
    return out
